# baseline (speedup 1.0000x reference)
import functools
import jax
import jax.numpy as jnp
from jax import lax
from jax.experimental import pallas as pl
from jax.experimental.pallas import tpu as pltpu

_THRESH = 0.7
_MIN_KEPT = 100000

_N, _C, _H, _W = 8, 19, 512, 512
_HW = _H * _W
_NPIX = _N * _HW
_BLK = 2048
_NSTEP = _HW // _BLK


_BH = 8


def _stats_body(pred_ref, tgt_ref, p_ref, nll_ref):
    for n in range(_N):
        x = pred_ref[n]
        t = tgt_ref[n]
        m = x[0]
        for c in range(1, _C):
            m = jnp.maximum(m, x[c])
        s = jnp.zeros_like(m)
        tl = jnp.zeros_like(m)
        for c in range(_C):
            xc = x[c]
            s = s + jnp.exp(xc - m)
            tl = tl + jnp.where(t == c, xc, 0.0)
        p_ref[n] = jnp.exp(tl - m) / s
        nll_ref[n] = (m - tl) + jnp.log(s)


def _i32_const(v):
    v &= 0xFFFFFFFF
    if v >= 1 << 31:
        v -= 1 << 32
    return jnp.int32(v)


def _select_body(p_ref, nll_ref, out_ref):
    ch = 16
    nch = _H // ch
    kf = jnp.float32(_MIN_KEPT)

    prefix = jnp.int32(0)
    k_rem = kf
    for shift in range(28, -1, -4):
        mask_above = _i32_const(0xFFFFFFFF << (shift + 4))

        def chunk(i, accs, shift=shift, mask_above=mask_above,
                  prefix=prefix):
            sl = p_ref[:, pl.ds(i * ch, ch), :]
            bits = lax.bitcast_convert_type(sl, jnp.int32)
            match = (bits & mask_above) == (prefix & mask_above)
            nib = lax.shift_right_logical(bits, jnp.int32(shift)) & 15
            out = []
            for b in range(16):
                oh = jnp.where(match & (nib == b), 1.0, 0.0)
                a = accs[b]
                for j in range(ch):
                    a = a + oh[:, j, :]
                out.append(a)
            return tuple(out)

        accs = tuple(jnp.zeros((_N, _W), jnp.float32) for _ in range(16))
        accs = lax.fori_loop(0, nch, chunk, accs)
        cnts = [jnp.sum(a) for a in accs]

        cumb = jnp.float32(0.0)
        sel_b = jnp.int32(15)
        sel_cumb = jnp.float32(0.0)
        found = jnp.bool_(False)
        for b in range(16):
            hit = jnp.logical_and(jnp.logical_not(found),
                                  cumb + cnts[b] >= k_rem)
            sel_b = jnp.where(hit, jnp.int32(b), sel_b)
            sel_cumb = jnp.where(hit, cumb, sel_cumb)
            found = jnp.logical_or(found, hit)
            cumb = cumb + cnts[b]
        prefix = prefix | lax.shift_left(sel_b, jnp.int32(shift))
        k_rem = k_rem - sel_cumb

    thr = jnp.maximum(lax.bitcast_convert_type(prefix, jnp.float32),
                      jnp.float32(_THRESH))

    def red(i, carry):
        s_nll, s_cnt = carry
        pv = p_ref[:, pl.ds(i * ch, ch), :]
        nv = nll_ref[:, pl.ds(i * ch, ch), :]
        kept = pv <= thr
        s_nll = s_nll + jnp.sum(jnp.where(kept, nv, 0.0))
        s_cnt = s_cnt + jnp.sum(jnp.where(kept, 1.0, 0.0))
        return s_nll, s_cnt

    s_nll, s_cnt = lax.fori_loop(
        0, nch, red, (jnp.float32(0.0), jnp.float32(0.0)))
    out_ref[...] = (s_nll / jnp.maximum(s_cnt, 1.0)) * jnp.ones(
        (1, 1), jnp.float32)


def kernel(pred, target):
    p, nll = pl.pallas_call(
        _stats_body,
        grid=(_H // _BH,),
        in_specs=[
            pl.BlockSpec((_N, _C, _BH, _W), lambda i: (0, 0, i, 0)),
            pl.BlockSpec((_N, _BH, _W), lambda i: (0, i, 0)),
        ],
        out_specs=[
            pl.BlockSpec((_N, _BH, _W), lambda i: (0, i, 0)),
            pl.BlockSpec((_N, _BH, _W), lambda i: (0, i, 0)),
        ],
        out_shape=[
            jax.ShapeDtypeStruct((_N, _H, _W), jnp.float32),
            jax.ShapeDtypeStruct((_N, _H, _W), jnp.float32),
        ],
    )(pred, target)

    loss = pl.pallas_call(
        _select_body,
        in_specs=[
            pl.BlockSpec((_N, _H, _W), lambda: (0, 0, 0)),
            pl.BlockSpec((_N, _H, _W), lambda: (0, 0, 0)),
        ],
        out_specs=pl.BlockSpec((1, 1), lambda: (0, 0)),
        out_shape=jax.ShapeDtypeStruct((1, 1), jnp.float32),
    )(p, nll)
    return loss.reshape(())


# speedup vs baseline: 3.6620x; 3.6620x over previous
import functools
import jax
import jax.numpy as jnp
from jax import lax
from jax.experimental import pallas as pl
from jax.experimental.pallas import tpu as pltpu

_THRESH = 0.7
_MIN_KEPT = 100000

_N, _C, _H, _W = 8, 19, 512, 512
_HW = _H * _W
_NPIX = _N * _HW
_BLK = 2048
_NSTEP = _HW // _BLK


_BH = 8


def _stats_body(pred_ref, tgt_ref, p_ref, nll_ref):
    for n in range(_N):
        x = pred_ref[n]
        t = tgt_ref[n]
        m = x[0]
        for c in range(1, _C):
            m = jnp.maximum(m, x[c])
        s = jnp.zeros_like(m)
        tl = jnp.zeros_like(m)
        for c in range(_C):
            xc = x[c]
            s = s + jnp.exp(xc - m)
            tl = tl + jnp.where(t == c, xc, 0.0)
        p_ref[pl.ds(n * _BH, _BH), :] = jnp.exp(tl - m) / s
        nll_ref[pl.ds(n * _BH, _BH), :] = (m - tl) + jnp.log(s)


def _i32_const(v):
    v &= 0xFFFFFFFF
    if v >= 1 << 31:
        v -= 1 << 32
    return jnp.int32(v)


def _select_body(p_ref, nll_ref, out_ref):
    ch = 128
    nrows = _N * _H
    nch = nrows // ch
    kf = jnp.float32(_MIN_KEPT)

    prefix = jnp.int32(0)
    k_rem = kf
    for shift in range(28, -1, -4):
        mask_above = _i32_const(0xFFFFFFFF << (shift + 4))

        def chunk(i, accs, shift=shift, mask_above=mask_above,
                  prefix=prefix):
            sl = p_ref[pl.ds(i * ch, ch), :]
            bits = lax.bitcast_convert_type(sl, jnp.int32)
            match = (bits & mask_above) == (prefix & mask_above)
            nib = lax.shift_right_logical(bits, jnp.int32(shift)) & 15
            out = []
            for b in range(16):
                oh = jnp.where(match & (nib == b), 1.0, 0.0)
                a = accs[b]
                for j in range(ch // 8):
                    a = a + oh[j * 8:(j + 1) * 8, :]
                out.append(a)
            return tuple(out)

        accs = tuple(jnp.zeros((8, _W), jnp.float32) for _ in range(16))
        accs = lax.fori_loop(0, nch, chunk, accs)
        cnts = [jnp.sum(a) for a in accs]

        cumb = jnp.float32(0.0)
        sel_b = jnp.int32(15)
        sel_cumb = jnp.float32(0.0)
        found = jnp.bool_(False)
        for b in range(16):
            hit = jnp.logical_and(jnp.logical_not(found),
                                  cumb + cnts[b] >= k_rem)
            sel_b = jnp.where(hit, jnp.int32(b), sel_b)
            sel_cumb = jnp.where(hit, cumb, sel_cumb)
            found = jnp.logical_or(found, hit)
            cumb = cumb + cnts[b]
        prefix = prefix | lax.shift_left(sel_b, jnp.int32(shift))
        k_rem = k_rem - sel_cumb

    thr = jnp.maximum(lax.bitcast_convert_type(prefix, jnp.float32),
                      jnp.float32(_THRESH))

    def red(i, carry):
        s_nll, s_cnt = carry
        pv = p_ref[pl.ds(i * ch, ch), :]
        nv = nll_ref[pl.ds(i * ch, ch), :]
        kept = pv <= thr
        s_nll = s_nll + jnp.sum(jnp.where(kept, nv, 0.0))
        s_cnt = s_cnt + jnp.sum(jnp.where(kept, 1.0, 0.0))
        return s_nll, s_cnt

    s_nll, s_cnt = lax.fori_loop(
        0, nch, red, (jnp.float32(0.0), jnp.float32(0.0)))
    out_ref[...] = (s_nll / jnp.maximum(s_cnt, 1.0)) * jnp.ones(
        (1, 1), jnp.float32)


def kernel(pred, target):
    p, nll = pl.pallas_call(
        _stats_body,
        grid=(_H // _BH,),
        in_specs=[
            pl.BlockSpec((_N, _C, _BH, _W), lambda i: (0, 0, i, 0)),
            pl.BlockSpec((_N, _BH, _W), lambda i: (0, i, 0)),
        ],
        out_specs=[
            pl.BlockSpec((_N * _BH, _W), lambda i: (i, 0)),
            pl.BlockSpec((_N * _BH, _W), lambda i: (i, 0)),
        ],
        out_shape=[
            jax.ShapeDtypeStruct((_N * _H, _W), jnp.float32),
            jax.ShapeDtypeStruct((_N * _H, _W), jnp.float32),
        ],
    )(pred, target)

    loss = pl.pallas_call(
        _select_body,
        in_specs=[
            pl.BlockSpec((_N * _H, _W), lambda: (0, 0)),
            pl.BlockSpec((_N * _H, _W), lambda: (0, 0)),
        ],
        out_specs=pl.BlockSpec((1, 1), lambda: (0, 0)),
        out_shape=jax.ShapeDtypeStruct((1, 1), jnp.float32),
    )(p, nll)
    return loss.reshape(())


# speedup vs baseline: 4.5773x; 1.2499x over previous
import functools
import jax
import jax.numpy as jnp
from jax import lax
from jax.experimental import pallas as pl
from jax.experimental.pallas import tpu as pltpu

_THRESH = 0.7
_MIN_KEPT = 100000

_N, _C, _H, _W = 8, 19, 512, 512
_HW = _H * _W
_NPIX = _N * _HW
_BLK = 2048
_NSTEP = _HW // _BLK


_BH = 8


def _stats_body(pred_ref, tgt_ref, p_ref, nll_ref):
    for n in range(_N):
        x = pred_ref[n]
        t = tgt_ref[n]
        m = x[0]
        for c in range(1, _C):
            m = jnp.maximum(m, x[c])
        s = jnp.zeros_like(m)
        tl = jnp.zeros_like(m)
        for c in range(_C):
            xc = x[c]
            s = s + jnp.exp(xc - m)
            tl = tl + jnp.where(t == c, xc, 0.0)
        p_ref[pl.ds(n * _BH, _BH), :] = jnp.exp(tl - m) / s
        nll_ref[pl.ds(n * _BH, _BH), :] = (m - tl) + jnp.log(s)


def _i32_const(v):
    v &= 0xFFFFFFFF
    if v >= 1 << 31:
        v -= 1 << 32
    return jnp.int32(v)


def _select_body(p_ref, nll_ref, out_ref):
    ch = 128
    nrows = _N * _H
    nch = nrows // ch
    kf = jnp.float32(_MIN_KEPT)

    prefix = jnp.int32(0)
    k_rem = kf
    for shift in range(28, -1, -4):
        nbin = 4 if shift == 28 else 16
        mask_ge = _i32_const(0xFFFFFFFF << shift)

        def chunk(i, accs, shift=shift, mask_ge=mask_ge, prefix=prefix,
                  nbin=nbin):
            sl = p_ref[pl.ds(i * ch, ch), :]
            bits = lax.bitcast_convert_type(sl, jnp.int32)
            y = bits & mask_ge
            out = []
            for b in range(nbin):
                tgt = prefix | lax.shift_left(jnp.int32(b), jnp.int32(shift))
                oh = jnp.where(y == tgt, 1.0, 0.0)
                a = accs[b]
                for j in range(ch // 8):
                    a = a + oh[j * 8:(j + 1) * 8, :]
                out.append(a)
            return tuple(out)

        accs = tuple(jnp.zeros((8, _W), jnp.float32) for _ in range(nbin))
        accs = lax.fori_loop(0, nch, chunk, accs)
        cnts = [jnp.sum(a) for a in accs]

        cumb = jnp.float32(0.0)
        sel_b = jnp.int32(nbin - 1)
        sel_cumb = jnp.float32(0.0)
        found = jnp.bool_(False)
        for b in range(nbin):
            hit = jnp.logical_and(jnp.logical_not(found),
                                  cumb + cnts[b] >= k_rem)
            sel_b = jnp.where(hit, jnp.int32(b), sel_b)
            sel_cumb = jnp.where(hit, cumb, sel_cumb)
            found = jnp.logical_or(found, hit)
            cumb = cumb + cnts[b]
        prefix = prefix | lax.shift_left(sel_b, jnp.int32(shift))
        k_rem = k_rem - sel_cumb

    thr = jnp.maximum(lax.bitcast_convert_type(prefix, jnp.float32),
                      jnp.float32(_THRESH))

    def red(i, carry):
        s_nll, s_cnt = carry
        pv = p_ref[pl.ds(i * ch, ch), :]
        nv = nll_ref[pl.ds(i * ch, ch), :]
        kept = pv <= thr
        s_nll = s_nll + jnp.sum(jnp.where(kept, nv, 0.0))
        s_cnt = s_cnt + jnp.sum(jnp.where(kept, 1.0, 0.0))
        return s_nll, s_cnt

    s_nll, s_cnt = lax.fori_loop(
        0, nch, red, (jnp.float32(0.0), jnp.float32(0.0)))
    out_ref[...] = (s_nll / jnp.maximum(s_cnt, 1.0)) * jnp.ones(
        (1, 1), jnp.float32)


def kernel(pred, target):
    p, nll = pl.pallas_call(
        _stats_body,
        grid=(_H // _BH,),
        in_specs=[
            pl.BlockSpec((_N, _C, _BH, _W), lambda i: (0, 0, i, 0)),
            pl.BlockSpec((_N, _BH, _W), lambda i: (0, i, 0)),
        ],
        out_specs=[
            pl.BlockSpec((_N * _BH, _W), lambda i: (i, 0)),
            pl.BlockSpec((_N * _BH, _W), lambda i: (i, 0)),
        ],
        out_shape=[
            jax.ShapeDtypeStruct((_N * _H, _W), jnp.float32),
            jax.ShapeDtypeStruct((_N * _H, _W), jnp.float32),
        ],
    )(pred, target)

    loss = pl.pallas_call(
        _select_body,
        in_specs=[
            pl.BlockSpec((_N * _H, _W), lambda: (0, 0)),
            pl.BlockSpec((_N * _H, _W), lambda: (0, 0)),
        ],
        out_specs=pl.BlockSpec((1, 1), lambda: (0, 0)),
        out_shape=jax.ShapeDtypeStruct((1, 1), jnp.float32),
    )(p, nll)
    return loss.reshape(())
